# baseline (speedup 1.0000x reference)
import functools

import jax
import jax.numpy as jnp
from jax import lax
from jax.experimental import pallas as pl
from jax.experimental.pallas import tpu as pltpu
from jax.experimental.pallas import tpu_sc as plsc

_B = 16384
_D = 32
_NC = 2
_NS = 16
_NW = _NC * _NS
_BPW = _B // _NW
_CH = 128
_NCH = _BPW // _CH
_L = 16
_G = _BPW // _L


def _cf_body(uid_hbm, iid_hbm, uemb_hbm, iemb_hbm, out_hbm,
             uidx, iidx, urows, irows, outv, sem):
    wid = lax.axis_index("s") * _NC + lax.axis_index("c")

    base = wid * _BPW

    for i in range(_NCH):
        pltpu.sync_copy(uid_hbm.at[pl.ds(base + i * _CH, _CH)], uidx.at[i])
        pltpu.sync_copy(iid_hbm.at[pl.ds(base + i * _CH, _CH)], iidx.at[i])

    copies = []
    for i in range(_NCH):
        copies.append(pltpu.async_copy(
            uemb_hbm.at[uidx.at[i]], urows.at[pl.ds(i * _CH, _CH)], sem))
        copies.append(pltpu.async_copy(
            iemb_hbm.at[iidx.at[i]], irows.at[pl.ds(i * _CH, _CH)], sem))
    for c in copies:
        c.wait()

    lane = lax.iota(jnp.int32, _L)

    def group(g, carry):
        r0 = g * _L
        acc = jnp.zeros((_L,), jnp.float32)
        for k in range(_L):
            j = r0 + k
            u1 = urows[j, pl.ds(0, _L)]
            u2 = urows[j, pl.ds(_L, _L)]
            v1 = irows[j, pl.ds(0, _L)]
            v2 = irows[j, pl.ds(_L, _L)]
            p = u1 * v1 + u2 * v2
            acc = jnp.where(lane == k, jnp.sum(p), acc)
        outv[pl.ds(r0, _L)] = 1.0 / (1.0 + jnp.exp(-acc))
        return carry

    lax.fori_loop(0, _G, group, 0)

    pltpu.sync_copy(outv, out_hbm.at[pl.ds(wid * _BPW, _BPW)])


_cf_call = functools.partial(
    pl.kernel,
    out_type=jax.ShapeDtypeStruct((_B,), jnp.float32),
    mesh=plsc.VectorSubcoreMesh(core_axis_name="c", subcore_axis_name="s"),
    compiler_params=pltpu.CompilerParams(needs_layout_passes=False, use_tc_tiling_on_sc=False),
    scratch_types=[
        pltpu.VMEM((_NCH, _CH), jnp.int32),
        pltpu.VMEM((_NCH, _CH), jnp.int32),
        pltpu.VMEM((_BPW, _D), jnp.float32),
        pltpu.VMEM((_BPW, _D), jnp.float32),
        pltpu.VMEM((_BPW,), jnp.float32),
        pltpu.SemaphoreType.DMA,
    ],
)(_cf_body)


def kernel(user_ids, item_ids, user_emb, item_emb):
    return _cf_call(user_ids, item_ids, user_emb, item_emb)


# speedup vs baseline: 4.8007x; 4.8007x over previous
import functools

import jax
import jax.numpy as jnp
from jax import lax
from jax.experimental import pallas as pl
from jax.experimental.pallas import tpu as pltpu
from jax.experimental.pallas import tpu_sc as plsc

_B = 16384
_D = 32
_NC = 2
_NS = 16
_NW = _NC * _NS
_BPW = _B // _NW
_L = 16
_G = _BPW // _L
_DB = 4
_SL = 8
_W = 16
_NCLS = 128 // _W
_SLOTS = 8


def _issue_window(emb_hbm, uid, win, slot_off, voff, sem):
    base128 = pl.multiple_of(uid & -128, 128)
    cls = (uid >> 4) & (_NCLS - 1)
    dst = win.at[:, :, pl.ds(slot_off + voff, _W)]

    def mk(j):
        def br():
            pltpu.async_copy(
                emb_hbm.at[:, :, pl.ds(base128 + j * _W, _W)], dst, sem)
            return jnp.int32(0)
        return br

    lax.switch(cls, tuple(mk(j) for j in range(_NCLS)))


def _cf_body(uid_hbm, iid_hbm, uemb_hbm, iemb_hbm, out_hbm,
             uidx, iidx, win1, win2, outv, sem):
    wid = lax.axis_index("s") * _NC + lax.axis_index("c")
    base = wid * _BPW

    pltpu.sync_copy(uid_hbm.at[pl.ds(base, _BPW)], uidx.at[pl.ds(0, _BPW)])
    pltpu.sync_copy(iid_hbm.at[pl.ds(base, _BPW)], iidx.at[pl.ds(0, _BPW)])

    i16 = lax.iota(jnp.int32, _L)
    d_hi = i16 // _SL
    d_lo = i16 % _SL
    lane = i16

    def group(g, carry):
        i0 = g * _L

        def issue(win, kbase):
            def body(k, c):
                p = i0 + kbase + k
                uid = uidx[pl.ds(p, _L)][0]
                iid = iidx[pl.ds(p, _L)][0]
                soff = pl.multiple_of(k * 128, 128)
                _issue_window(uemb_hbm, uid, win, soff, 0, sem)
                _issue_window(iemb_hbm, iid, win, soff, 64, sem)
                return c
            lax.fori_loop(0, _SLOTS, body, 0)

        def drain(win):
            def body(k, c):
                soff = pl.multiple_of(k * 128, 128)
                for voff in (0, 64):
                    pltpu.make_async_copy(
                        uemb_hbm.at[:, :, pl.ds(0, _W)],
                        win.at[:, :, pl.ds(soff + voff, _W)], sem).wait()
                return c
            lax.fori_loop(0, _SLOTS, body, 0)

        def extract(win, kbase, acc0):
            def body(k, acc):
                p = i0 + kbase + k
                uid = uidx[pl.ds(p, _L)][0]
                iid = iidx[pl.ds(p, _L)][0]
                ul = jnp.full((_L,), k * 128 + (uid & (_W - 1)), jnp.int32)
                il = jnp.full((_L,), k * 128 + 64 + (iid & (_W - 1)), jnp.int32)
                cu0 = plsc.load_gather(win, [d_hi, d_lo, ul])
                cu1 = plsc.load_gather(win, [d_hi + 2, d_lo, ul])
                cv0 = plsc.load_gather(win, [d_hi, d_lo, il])
                cv1 = plsc.load_gather(win, [d_hi + 2, d_lo, il])
                s = jnp.sum(cu0 * cv0 + cu1 * cv1)
                return jnp.where(lane == kbase + k, s, acc)
            return lax.fori_loop(0, _SLOTS, body, acc0)

        issue(win1, 0)
        issue(win2, _SLOTS)
        drain(win1)
        drain(win2)
        acc = extract(win1, 0, jnp.zeros((_L,), jnp.float32))
        acc = extract(win2, _SLOTS, acc)
        outv[pl.ds(i0, _L)] = 1.0 / (1.0 + jnp.exp(-acc))
        return carry

    lax.fori_loop(0, _G, group, 0)

    pltpu.sync_copy(outv, out_hbm.at[pl.ds(base, _BPW)])


_cf_call = functools.partial(
    pl.kernel,
    out_type=jax.ShapeDtypeStruct((_B,), jnp.float32),
    mesh=plsc.VectorSubcoreMesh(core_axis_name="c", subcore_axis_name="s"),
    compiler_params=pltpu.CompilerParams(needs_layout_passes=False),
    scratch_types=[
        pltpu.VMEM((_BPW + _L,), jnp.int32),
        pltpu.VMEM((_BPW + _L,), jnp.int32),
        pltpu.VMEM((_DB, _SL, _SLOTS * 128), jnp.float32),
        pltpu.VMEM((_DB, _SL, _SLOTS * 128), jnp.float32),
        pltpu.VMEM((_BPW,), jnp.float32),
        pltpu.SemaphoreType.DMA,
    ],
)(_cf_body)


def kernel(user_ids, item_ids, user_emb, item_emb):
    ue = user_emb.T.reshape(_DB, _SL, user_emb.shape[0])
    ie = item_emb.T.reshape(_DB, _SL, item_emb.shape[0])
    return _cf_call(user_ids, item_ids, ue, ie)


# speedup vs baseline: 8.4864x; 1.7677x over previous
import functools

import jax
import jax.numpy as jnp
from jax import lax
from jax.experimental import pallas as pl
from jax.experimental.pallas import tpu as pltpu
from jax.experimental.pallas import tpu_sc as plsc

_B = 16384
_D = 32
_NC = 2
_NS = 16
_NW = _NC * _NS
_BPW = _B // _NW
_L = 16
_G = _BPW // _L
_DB = 4
_SL = 8
_W = 16
_NCLS = 128 // _W
_SLOTS = 8


def _issue_window(emb_hbm, uid, win, slot_off, voff, sem):
    base128 = pl.multiple_of(uid & -128, 128)
    cls = (uid >> 4) & (_NCLS - 1)
    dst = win.at[:, :, pl.ds(slot_off + voff, _W)]

    def mk(j):
        def br():
            pltpu.async_copy(
                emb_hbm.at[:, :, pl.ds(base128 + j * _W, _W)], dst, sem)
            return jnp.int32(0)
        return br

    lax.switch(cls, tuple(mk(j) for j in range(_NCLS)))


def _cf_body(uid_hbm, iid_hbm, uemb_hbm, iemb_hbm, out_hbm,
             uidx, iidx, win, outv, sem):
    wid = lax.axis_index("s") * _NC + lax.axis_index("c")
    base = wid * _BPW

    pltpu.sync_copy(uid_hbm.at[pl.ds(base, _BPW)], uidx.at[pl.ds(0, _BPW)])
    pltpu.sync_copy(iid_hbm.at[pl.ds(base, _BPW)], iidx.at[pl.ds(0, _BPW)])

    i16 = lax.iota(jnp.int32, _L)
    d_hi = i16 // _SL
    d_lo = i16 % _SL
    lane = i16

    slot128 = i16 * 128

    def group(g, carry):
        i0 = g * _L

        def issue(k, c):
            p = i0 + k
            uid = uidx[pl.ds(p, _L)][0]
            iid = iidx[pl.ds(p, _L)][0]
            soff = pl.multiple_of(k * 128, 128)
            _issue_window(uemb_hbm, uid, win, soff, 0, sem)
            _issue_window(iemb_hbm, iid, win, soff, 64, sem)
            return c
        lax.fori_loop(0, _L, issue, 0)

        def drain(k, c):
            soff = pl.multiple_of(k * 128, 128)
            for voff in (0, 64):
                pltpu.make_async_copy(
                    uemb_hbm.at[:, :, pl.ds(0, _W)],
                    win.at[:, :, pl.ds(soff + voff, _W)], sem).wait()
            return c
        lax.fori_loop(0, _L, drain, 0)

        uvec = uidx[pl.ds(i0, _L)]
        ivec = iidx[pl.ds(i0, _L)]
        uix = slot128 + (uvec & (_W - 1))
        vix = slot128 + 64 + (ivec & (_W - 1))
        acc = jnp.zeros((_L,), jnp.float32)
        for db in range(_DB):
            dbf = jnp.full((_L,), db, jnp.int32)
            for dl in range(_SL):
                dlf = jnp.full((_L,), dl, jnp.int32)
                acc = acc + (plsc.load_gather(win, [dbf, dlf, uix]) *
                             plsc.load_gather(win, [dbf, dlf, vix]))
        outv[pl.ds(i0, _L)] = 1.0 / (1.0 + jnp.exp(-acc))
        return carry

    lax.fori_loop(0, _G, group, 0)

    pltpu.sync_copy(outv, out_hbm.at[pl.ds(base, _BPW)])


_cf_call = functools.partial(
    pl.kernel,
    out_type=jax.ShapeDtypeStruct((_B,), jnp.float32),
    mesh=plsc.VectorSubcoreMesh(core_axis_name="c", subcore_axis_name="s"),
    compiler_params=pltpu.CompilerParams(needs_layout_passes=False),
    scratch_types=[
        pltpu.VMEM((_BPW + _L,), jnp.int32),
        pltpu.VMEM((_BPW + _L,), jnp.int32),
        pltpu.VMEM((_DB, _SL, _L * 128), jnp.float32),
        pltpu.VMEM((_BPW,), jnp.float32),
        pltpu.SemaphoreType.DMA,
    ],
)(_cf_body)


def kernel(user_ids, item_ids, user_emb, item_emb):
    ue = user_emb.T.reshape(_DB, _SL, user_emb.shape[0])
    ie = item_emb.T.reshape(_DB, _SL, item_emb.shape[0])
    return _cf_call(user_ids, item_ids, ue, ie)
